# baseline (speedup 1.0000x reference)
import functools

import jax
import jax.numpy as jnp
from jax import lax
from jax.experimental import pallas as pl
from jax.experimental.pallas import tpu as pltpu
from jax.experimental.pallas import tpu_sc as plsc

_TOKENS = 4096


def _conv2d(x, w, b=None, stride=1, padding=0):
    out = lax.conv_general_dilated(
        x, w, (stride, stride), ((padding, padding), (padding, padding)),
        dimension_numbers=('NCHW', 'OIHW', 'NCHW'))
    if b is not None:
        out = out + b[None, :, None, None]
    return out


def _conv_transpose2d(x, w, b, stride, padding):
    k = w.shape[2]
    w_t = jnp.transpose(w[:, :, ::-1, ::-1], (1, 0, 2, 3))
    pad = k - 1 - padding
    out = lax.conv_general_dilated(
        x, w_t, (1, 1), ((pad, pad), (pad, pad)), lhs_dilation=(stride, stride),
        dimension_numbers=('NCHW', 'OIHW', 'NCHW'))
    return out + b[None, :, None, None]


def _res_stack(x, layers):
    for (w1, w2) in layers:
        y = jax.nn.relu(x)
        y = _conv2d(y, w1, None, 1, 1)
        y = jax.nn.relu(y)
        y = _conv2d(y, w2, None, 1, 0)
        x = x + y
    return jax.nn.relu(x)


def _vq_small_body(zp_ref, tbl_ref, idx_ref, md_ref, m_sc, a_sc, *, d, n, T):
    ids0 = lax.broadcasted_iota(jnp.int32, (1, 128), 1)

    def tt_body(tt, _):
        zk = [zp_ref[pl.ds(tt * 8, 8), k:k + 1] for k in range(d)]
        m = jnp.full((8, 128), jnp.inf, jnp.float32)
        a = jnp.zeros((8, 128), jnp.int32)
        for g in range(n // 128):
            s = tbl_ref[d:d + 1, g * 128:(g + 1) * 128]
            for k in range(d):
                s = s + tbl_ref[k:k + 1, g * 128:(g + 1) * 128] * zk[k]
            upd = s < m
            m = jnp.where(upd, s, m)
            a = jnp.where(upd, jnp.broadcast_to(ids0 + g * 128, (8, 128)), a)
        m_sc[pl.ds(tt * 8, 8), :] = m
        a_sc[pl.ds(tt * 8, 8), :] = a
        return 0

    lax.fori_loop(0, T // 8, tt_body, 0)
    mm = m_sc[...]
    aa = a_sc[...]
    m = jnp.min(mm, axis=1)
    a = jnp.min(jnp.where(mm == m[:, None], aa, 2 ** 30), axis=1)
    zp = zp_ref[...]
    idx_ref[...] = a
    md_ref[...] = m + jnp.sum(zp * zp, axis=1)


def _vq_large_body(zT_ref, em2_ref, e2_ref, idx_ref, md_ref, bmin_ref,
                   barg_ref, *, d, C, nchunks):
    c = pl.program_id(0)
    zT = zT_ref[...]
    s = jnp.dot(em2_ref[...], zT, preferred_element_type=jnp.float32)
    s = s + e2_ref[...]
    m = jnp.min(s, axis=0)
    ii = lax.broadcasted_iota(jnp.int32, s.shape, 0)
    a = jnp.min(jnp.where(s == m[None, :], ii, 2 ** 30), axis=0) + c * C

    @pl.when(c == 0)
    def _():
        bmin_ref[...] = m
        barg_ref[...] = a

    @pl.when(c > 0)
    def _():
        pm = bmin_ref[...]
        upd = m < pm
        bmin_ref[...] = jnp.where(upd, m, pm)
        barg_ref[...] = jnp.where(upd, a, barg_ref[...])

    @pl.when(c == nchunks - 1)
    def _():
        idx_ref[...] = barg_ref[...]
        md_ref[...] = bmin_ref[...] + jnp.sum(zT * zT, axis=0)


def _vq_argmin(flat, emb):
    n, d = emb.shape
    if d <= 8:
        zp = jnp.pad(flat, ((0, 0), (0, 128 - d)))
        tbl = jnp.concatenate(
            [-2.0 * emb.T, jnp.sum(emb * emb, axis=1)[None, :]], axis=0)
        T = 1024
        grid = (_TOKENS // T,)
        body = functools.partial(_vq_small_body, d=d, n=n, T=T)
        return pl.pallas_call(
            body,
            grid=grid,
            in_specs=[
                pl.BlockSpec((T, 128), lambda t: (t, 0)),
                pl.BlockSpec((d + 1, n), lambda t: (0, 0)),
            ],
            out_specs=[
                pl.BlockSpec((T,), lambda t: (t,)),
                pl.BlockSpec((T,), lambda t: (t,)),
            ],
            out_shape=[
                jax.ShapeDtypeStruct((_TOKENS,), jnp.int32),
                jax.ShapeDtypeStruct((_TOKENS,), jnp.float32),
            ],
            scratch_shapes=[
                pltpu.VMEM((T, 128), jnp.float32),
                pltpu.VMEM((T, 128), jnp.int32),
            ],
        )(zp, tbl)
    zT = flat.T
    em2 = -2.0 * emb
    e2 = jnp.sum(emb * emb, axis=1)[:, None]
    C = min(n, 512)
    nchunks = n // C
    body = functools.partial(_vq_large_body, d=d, C=C, nchunks=nchunks)
    return pl.pallas_call(
        body,
        grid=(nchunks,),
        in_specs=[
            pl.BlockSpec((d, _TOKENS), lambda c: (0, 0)),
            pl.BlockSpec((C, d), lambda c: (c, 0)),
            pl.BlockSpec((C, 1), lambda c: (c, 0)),
        ],
        out_specs=[
            pl.BlockSpec((_TOKENS,), lambda c: (0,)),
            pl.BlockSpec((_TOKENS,), lambda c: (0,)),
        ],
        out_shape=[
            jax.ShapeDtypeStruct((_TOKENS,), jnp.int32),
            jax.ShapeDtypeStruct((_TOKENS,), jnp.float32),
        ],
        scratch_shapes=[
            pltpu.VMEM((_TOKENS,), jnp.float32),
            pltpu.VMEM((_TOKENS,), jnp.int32),
        ],
    )(zT, em2, e2)


def _vq1_sc(z1, es):
    NB = _TOKENS // 32
    mesh = plsc.VectorSubcoreMesh(core_axis_name="c", subcore_axis_name="s")

    @functools.partial(
        pl.kernel, mesh=mesh,
        compiler_params=pltpu.CompilerParams(use_tc_tiling_on_sc=False,
                                             needs_layout_passes=False),
        out_type=[jax.ShapeDtypeStruct((_TOKENS,), jnp.float32),
                  jax.ShapeDtypeStruct((_TOKENS,), jnp.float32)],
        scratch_types=[pltpu.VMEM((65536,), jnp.float32),
                       pltpu.VMEM((NB,), jnp.float32),
                       pltpu.VMEM((NB,), jnp.float32),
                       pltpu.VMEM((NB,), jnp.float32)],
    )
    def k(z_hbm, es_hbm, q_hbm, md_hbm, tab_v, z_v, q_v, md_v):
        wid = lax.axis_index("s") * 2 + lax.axis_index("c")
        base = wid * NB
        pltpu.sync_copy(es_hbm, tab_v)
        pltpu.sync_copy(z_hbm.at[pl.ds(base, NB)], z_v)
        for v in range(NB // 16):
            zv = z_v[pl.ds(v * 16, 16)]
            pos = jnp.full((16,), -1, jnp.int32)
            for shift in range(15, -1, -1):
                nxt = pos + (1 << shift)
                val = plsc.load_gather(tab_v, [nxt])
                pos = jnp.where(val < zv, nxt, pos)
            cl = jnp.maximum(pos, 0)
            cr = jnp.minimum(pos + 1, 65535)
            el = plsc.load_gather(tab_v, [cl])
            er = plsc.load_gather(tab_v, [cr])
            dl = el * el - 2.0 * zv * el
            dr = er * er - 2.0 * zv * er
            ch = dr < dl
            q_v[pl.ds(v * 16, 16)] = jnp.where(ch, er, el)
            md_v[pl.ds(v * 16, 16)] = jnp.where(ch, dr, dl) + zv * zv
        pltpu.sync_copy(q_v, q_hbm.at[pl.ds(base, NB)])
        pltpu.sync_copy(md_v, md_hbm.at[pl.ds(base, NB)])

    return k(z1, es)


def kernel(x, params, gumbel_u):
    p = params
    h = _conv2d(x, p['enc_w1'], p['enc_b1'], 2, 1)
    h = jax.nn.relu(h)
    h = _conv2d(h, p['enc_w2'], p['enc_b2'], 2, 1)
    h = jax.nn.relu(h)
    h = _conv2d(h, p['enc_w3'], p['enc_b3'], 1, 1)
    h = _res_stack(h, [(p['enc_r1_w1'], p['enc_r1_w2']),
                       (p['enc_r2_w1'], p['enc_r2_w2'])])

    qf = h.reshape(-1, 128)
    N = qf.shape[0]
    qp = qf @ p['wq'].T + p['bq']
    kf = p['qkeys'].reshape(8, 128)
    kp = kf @ p['wk'].T + p['bk']
    qh = jnp.transpose(qp.reshape(N, 2, 64), (1, 0, 2))
    kh = jnp.transpose(kp.reshape(8, 2, 64), (1, 0, 2))
    scores = jnp.einsum('hqd,hkd->hqk', qh, kh) / jnp.sqrt(64.0)
    att = jnp.mean(jax.nn.softmax(scores, axis=-1), axis=0)[None]
    g = -jnp.log(-jnp.log(gumbel_u + 1e-20) + 1e-20)
    y_soft = jax.nn.softmax(att + g, axis=2)
    route = jnp.argmax(y_soft, axis=2)[0]

    h_t = jnp.transpose(h, (0, 2, 3, 1)).reshape(-1, 128)
    w_cat = jnp.concatenate([p['vq_mw'][i][:, :, 0, 0].T for i in range(8)],
                            axis=1)
    b_cat = jnp.concatenate([p['vq_mb'][i] for i in range(8)])
    flat_cat = h_t @ w_cat + b_cat

    cb_loss = 0.0
    qs = []
    off = 0
    for i in range(8):
        emb = p['vq_emb'][i]
        d = emb.shape[1]
        flat = flat_cat[:, off:off + d]
        off += d
        if d == 1:
            es = jnp.sort(emb[:, 0])
            qv, md = _vq1_sc(flat[:, 0], es)
            q = qv.reshape(64, 8, 8, 1)
        else:
            idx, md = _vq_argmin(flat, emb)
            q = jnp.take(emb, idx, axis=0).reshape(64, 8, 8, d)
        cb_loss = cb_loss + 1.25 * jnp.sum(md) / (_TOKENS * d)
        qs.append(q)
    extra_loss = cb_loss / 8.0

    q_cat = jnp.transpose(jnp.concatenate(qs, axis=3), (0, 3, 1, 2))
    w_bd = jnp.zeros((1024, 255, 3, 3), jnp.float32)
    off = 0
    for i in range(8):
        cw = p['vq_cw'][i]
        w_bd = lax.dynamic_update_slice(w_bd, cw, (128 * i, off, 0, 0))
        off += cw.shape[1]
    b_bd = jnp.concatenate([p['vq_cb'][i] for i in range(8)])
    nv_cat = _conv2d(q_cat, w_bd, b_bd, 1, 1)
    zs = [nv_cat[:, 128 * i:128 * (i + 1)].reshape(-1, 128) for i in range(8)]

    Zs = jnp.stack(zs, axis=1)
    state = jnp.take_along_axis(Zs, route[:, None, None], axis=1)[:, 0, :]
    state = state.reshape(64, 128, 8, 8)

    d_ = _res_stack(state, [(p['dec_r1_w1'], p['dec_r1_w2']),
                            (p['dec_r2_w1'], p['dec_r2_w2'])])
    d_ = _conv_transpose2d(d_, p['dec_t1_w'], p['dec_t1_b'], 2, 1)
    d_ = jax.nn.relu(d_)
    x_recon = _conv_transpose2d(d_, p['dec_t2_w'], p['dec_t2_b'], 2, 1)
    recon_loss = jnp.mean((x - x_recon) ** 2)
    loss = recon_loss + extra_loss
    return loss, x_recon


# speedup vs baseline: 1.1111x; 1.1111x over previous
import functools

import jax
import jax.numpy as jnp
from jax import lax
from jax.experimental import pallas as pl
from jax.experimental.pallas import tpu as pltpu
from jax.experimental.pallas import tpu_sc as plsc

_TOKENS = 4096


def _conv2d(x, w, b=None, stride=1, padding=0):
    out = lax.conv_general_dilated(
        x, w, (stride, stride), ((padding, padding), (padding, padding)),
        dimension_numbers=('NCHW', 'OIHW', 'NCHW'))
    if b is not None:
        out = out + b[None, :, None, None]
    return out


def _conv_transpose2d(x, w, b, stride, padding):
    k = w.shape[2]
    w_t = jnp.transpose(w[:, :, ::-1, ::-1], (1, 0, 2, 3))
    pad = k - 1 - padding
    out = lax.conv_general_dilated(
        x, w_t, (1, 1), ((pad, pad), (pad, pad)), lhs_dilation=(stride, stride),
        dimension_numbers=('NCHW', 'OIHW', 'NCHW'))
    return out + b[None, :, None, None]


def _res_stack(x, layers):
    for (w1, w2) in layers:
        y = jax.nn.relu(x)
        y = _conv2d(y, w1, None, 1, 1)
        y = jax.nn.relu(y)
        y = _conv2d(y, w2, None, 1, 0)
        x = x + y
    return jax.nn.relu(x)


def _vq_small_body(zp_ref, tbl_ref, idx_ref, md_ref, m_sc, a_sc, *, d, n, T):
    ids0 = lax.broadcasted_iota(jnp.int32, (1, 128), 1)

    def tt_body(tt, _):
        zk = [zp_ref[pl.ds(tt * 8, 8), k:k + 1] for k in range(d)]
        m = jnp.full((8, 128), jnp.inf, jnp.float32)
        a = jnp.zeros((8, 128), jnp.int32)
        for g in range(n // 128):
            s = tbl_ref[d:d + 1, g * 128:(g + 1) * 128]
            for k in range(d):
                s = s + tbl_ref[k:k + 1, g * 128:(g + 1) * 128] * zk[k]
            upd = s < m
            m = jnp.where(upd, s, m)
            a = jnp.where(upd, jnp.broadcast_to(ids0 + g * 128, (8, 128)), a)
        m_sc[pl.ds(tt * 8, 8), :] = m
        a_sc[pl.ds(tt * 8, 8), :] = a
        return 0

    lax.fori_loop(0, T // 8, tt_body, 0)
    mm = m_sc[...]
    aa = a_sc[...]
    m = jnp.min(mm, axis=1)
    a = jnp.min(jnp.where(mm == m[:, None], aa, 2 ** 30), axis=1)
    zp = zp_ref[...]
    idx_ref[...] = a
    md_ref[...] = m + jnp.sum(zp * zp, axis=1)


def _vq_large_body(zT_ref, em2_ref, e2_ref, idx_ref, md_ref, bmin_ref,
                   barg_ref, *, d, C, nchunks):
    c = pl.program_id(0)
    zT = zT_ref[...]
    s = jnp.dot(em2_ref[...], zT, preferred_element_type=jnp.float32)
    s = s + e2_ref[...]
    m = jnp.min(s, axis=0)
    ii = lax.broadcasted_iota(jnp.int32, s.shape, 0)
    a = jnp.min(jnp.where(s == m[None, :], ii, 2 ** 30), axis=0) + c * C

    @pl.when(c == 0)
    def _():
        bmin_ref[...] = m
        barg_ref[...] = a

    @pl.when(c > 0)
    def _():
        pm = bmin_ref[...]
        upd = m < pm
        bmin_ref[...] = jnp.where(upd, m, pm)
        barg_ref[...] = jnp.where(upd, a, barg_ref[...])

    @pl.when(c == nchunks - 1)
    def _():
        idx_ref[...] = barg_ref[...]
        md_ref[...] = bmin_ref[...] + jnp.sum(zT * zT, axis=0)


def _vq_argmin(flat, emb):
    n, d = emb.shape
    if d <= 8:
        zp = jnp.pad(flat, ((0, 0), (0, 128 - d)))
        tbl = jnp.concatenate(
            [-2.0 * emb.T, jnp.sum(emb * emb, axis=1)[None, :]], axis=0)
        T = 1024
        grid = (_TOKENS // T,)
        body = functools.partial(_vq_small_body, d=d, n=n, T=T)
        return pl.pallas_call(
            body,
            grid=grid,
            in_specs=[
                pl.BlockSpec((T, 128), lambda t: (t, 0)),
                pl.BlockSpec((d + 1, n), lambda t: (0, 0)),
            ],
            out_specs=[
                pl.BlockSpec((T,), lambda t: (t,)),
                pl.BlockSpec((T,), lambda t: (t,)),
            ],
            out_shape=[
                jax.ShapeDtypeStruct((_TOKENS,), jnp.int32),
                jax.ShapeDtypeStruct((_TOKENS,), jnp.float32),
            ],
            scratch_shapes=[
                pltpu.VMEM((T, 128), jnp.float32),
                pltpu.VMEM((T, 128), jnp.int32),
            ],
        )(zp, tbl)
    zT = flat.T
    em2 = -2.0 * emb
    e2 = jnp.sum(emb * emb, axis=1)[:, None]
    C = min(n, 512)
    nchunks = n // C
    body = functools.partial(_vq_large_body, d=d, C=C, nchunks=nchunks)
    return pl.pallas_call(
        body,
        grid=(nchunks,),
        in_specs=[
            pl.BlockSpec((d, _TOKENS), lambda c: (0, 0)),
            pl.BlockSpec((C, d), lambda c: (c, 0)),
            pl.BlockSpec((C, 1), lambda c: (c, 0)),
        ],
        out_specs=[
            pl.BlockSpec((_TOKENS,), lambda c: (0,)),
            pl.BlockSpec((_TOKENS,), lambda c: (0,)),
        ],
        out_shape=[
            jax.ShapeDtypeStruct((_TOKENS,), jnp.int32),
            jax.ShapeDtypeStruct((_TOKENS,), jnp.float32),
        ],
        scratch_shapes=[
            pltpu.VMEM((_TOKENS,), jnp.float32),
            pltpu.VMEM((_TOKENS,), jnp.int32),
        ],
    )(zT, em2, e2)


def _vq1_sc(z1, es):
    NB = _TOKENS // 32
    mesh = plsc.VectorSubcoreMesh(core_axis_name="c", subcore_axis_name="s")

    @functools.partial(
        pl.kernel, mesh=mesh,
        compiler_params=pltpu.CompilerParams(use_tc_tiling_on_sc=False,
                                             needs_layout_passes=False),
        out_type=[jax.ShapeDtypeStruct((_TOKENS,), jnp.float32),
                  jax.ShapeDtypeStruct((_TOKENS,), jnp.float32)],
        scratch_types=[pltpu.VMEM((65536,), jnp.float32),
                       pltpu.VMEM((NB,), jnp.float32),
                       pltpu.VMEM((NB,), jnp.float32),
                       pltpu.VMEM((NB,), jnp.float32)],
    )
    def k(z_hbm, es_hbm, q_hbm, md_hbm, tab_v, z_v, q_v, md_v):
        wid = lax.axis_index("s") * 2 + lax.axis_index("c")
        base = wid * NB
        pltpu.sync_copy(es_hbm, tab_v)
        pltpu.sync_copy(z_hbm.at[pl.ds(base, NB)], z_v)
        for v in range(NB // 16):
            zv = z_v[pl.ds(v * 16, 16)]
            pos = jnp.full((16,), -1, jnp.int32)
            for shift in range(15, -1, -1):
                nxt = pos + (1 << shift)
                val = plsc.load_gather(tab_v, [nxt])
                pos = jnp.where(val < zv, nxt, pos)
            cl = jnp.maximum(pos, 0)
            cr = jnp.minimum(pos + 1, 65535)
            el = plsc.load_gather(tab_v, [cl])
            er = plsc.load_gather(tab_v, [cr])
            dl = el * el - 2.0 * zv * el
            dr = er * er - 2.0 * zv * er
            ch = dr < dl
            q_v[pl.ds(v * 16, 16)] = jnp.where(ch, er, el)
            md_v[pl.ds(v * 16, 16)] = jnp.where(ch, dr, dl) + zv * zv
        pltpu.sync_copy(q_v, q_hbm.at[pl.ds(base, NB)])
        pltpu.sync_copy(md_v, md_hbm.at[pl.ds(base, NB)])

    return k(z1, es)


def kernel(x, params, gumbel_u):
    p = params
    h = _conv2d(x, p['enc_w1'], p['enc_b1'], 2, 1)
    h = jax.nn.relu(h)
    h = _conv2d(h, p['enc_w2'], p['enc_b2'], 2, 1)
    h = jax.nn.relu(h)
    h = _conv2d(h, p['enc_w3'], p['enc_b3'], 1, 1)
    h = _res_stack(h, [(p['enc_r1_w1'], p['enc_r1_w2']),
                       (p['enc_r2_w1'], p['enc_r2_w2'])])

    qf = h.reshape(-1, 128)
    N = qf.shape[0]
    qp = qf @ p['wq'].T + p['bq']
    kf = p['qkeys'].reshape(8, 128)
    kp = kf @ p['wk'].T + p['bk']
    qh = jnp.transpose(qp.reshape(N, 2, 64), (1, 0, 2))
    kh = jnp.transpose(kp.reshape(8, 2, 64), (1, 0, 2))
    scores = jnp.einsum('hqd,hkd->hqk', qh, kh) / jnp.sqrt(64.0)
    att = jnp.mean(jax.nn.softmax(scores, axis=-1), axis=0)[None]
    g = -jnp.log(-jnp.log(gumbel_u + 1e-20) + 1e-20)
    y_soft = jax.nn.softmax(att + g, axis=2)
    route = jnp.argmax(y_soft, axis=2)[0]

    h_t = jnp.transpose(h, (0, 2, 3, 1)).reshape(-1, 128)
    w_cat = jnp.concatenate([p['vq_mw'][i][:, :, 0, 0].T for i in range(8)],
                            axis=1)
    b_cat = jnp.concatenate([p['vq_mb'][i] for i in range(8)])
    flat_cat = h_t @ w_cat + b_cat

    cb_loss = 0.0
    zs = []
    off = 0
    for i in range(8):
        emb = p['vq_emb'][i]
        d = emb.shape[1]
        flat = flat_cat[:, off:off + d]
        off += d
        if d == 1:
            es = jnp.sort(emb[:, 0])
            qv, md = _vq1_sc(flat[:, 0], es)
            q = qv.reshape(64, 8, 8, 1)
        else:
            idx, md = _vq_argmin(flat, emb)
            q = jnp.take(emb, idx, axis=0).reshape(64, 8, 8, d)
        cb_loss = cb_loss + 1.25 * jnp.sum(md) / (_TOKENS * d)
        q_nchw = jnp.transpose(q, (0, 3, 1, 2))
        nv = _conv2d(q_nchw, p['vq_cw'][i], p['vq_cb'][i], 1, 1)
        zs.append(nv.reshape(-1, 128))
    extra_loss = cb_loss / 8.0

    Zs = jnp.stack(zs, axis=1)
    state = jnp.take_along_axis(Zs, route[:, None, None], axis=1)[:, 0, :]
    state = state.reshape(64, 128, 8, 8)

    d_ = _res_stack(state, [(p['dec_r1_w1'], p['dec_r1_w2']),
                            (p['dec_r2_w1'], p['dec_r2_w2'])])
    d_ = _conv_transpose2d(d_, p['dec_t1_w'], p['dec_t1_b'], 2, 1)
    d_ = jax.nn.relu(d_)
    x_recon = _conv_transpose2d(d_, p['dec_t2_w'], p['dec_t2_b'], 2, 1)
    recon_loss = jnp.mean((x - x_recon) ** 2)
    loss = recon_loss + extra_loss
    return loss, x_recon
